# baseline (speedup 1.0000x reference)
import functools

import jax
import jax.numpy as jnp
from jax import lax
from jax.experimental import pallas as pl
from jax.experimental.pallas import tpu as pltpu
from jax.experimental.pallas import tpu_sc as plsc

NUM_HEROES = 1000000
EMBED_DIM = 32
BATCH = 16384
SEQ = 10

_info = plsc.get_sparse_core_info()
NC, NS, NL = _info.num_cores, _info.num_subcores, _info.num_lanes
NW = NC * NS
NBT = BATCH // 128
BT_PER_W = NBT // NW
NFT = EMBED_DIM // 8
SEQ_PAD = 16


def _body(x4_hbm, table_hbm, out5, idx_c, buf_g, buf_t, gsem, wsem):
    wid = lax.axis_index("s") * NC + lax.axis_index("c")
    lanes = lax.iota(jnp.int32, NL)

    def chunk(c, carry):
        bt = BT_PER_W * wid + c
        pltpu.sync_copy(x4_hbm.at[0, bt], idx_c.at[pl.ds(0, 8)])
        pltpu.sync_copy(x4_hbm.at[1, bt, pl.ds(0, 2)], idx_c.at[pl.ds(8, 2)])

        descs = [
            pltpu.async_copy(table_hbm.at[idx_c.at[s]], buf_g.at[s], gsem)
            for s in range(SEQ)
        ]
        for d in descs:
            d.wait()

        def seq_step(s, carry2):
            s_vec = jnp.full((NL,), s, jnp.int32)
            for f in range(EMBED_DIM):
                f_vec = jnp.full((NL,), f, jnp.int32)
                for bg in range(128 // NL):
                    bl_vec = lanes + (bg * NL)
                    v = plsc.load_gather(buf_g, [s_vec, bl_vec, f_vec])
                    buf_t[s, f // 8, f % 8, pl.ds(bg * NL, NL)] = v
            for ft in range(NFT):
                pltpu.async_copy(buf_t.at[s, ft], out5.at[s, ft, bt],
                                 wsem).wait()
            return carry2

        lax.fori_loop(0, SEQ, seq_step, 0)
        return carry

    lax.fori_loop(0, BT_PER_W, chunk, 0)


@jax.jit
def kernel(x, table):
    xp = jnp.pad(x.T, ((0, SEQ_PAD - SEQ), (0, 0)))
    x4 = xp.reshape(2, 8, NBT, 128).transpose(0, 2, 1, 3)

    run = pl.kernel(
        _body,
        out_type=jax.ShapeDtypeStruct((SEQ, NFT, NBT, 8, 128), jnp.float32),
        mesh=plsc.VectorSubcoreMesh(core_axis_name="c", subcore_axis_name="s"),
        scratch_types=[
            pltpu.VMEM((SEQ, 128), jnp.int32),
            pltpu.VMEM((SEQ, 128, EMBED_DIM), jnp.float32),
            pltpu.VMEM((SEQ, NFT, 8, 128), jnp.float32),
            pltpu.SemaphoreType.DMA,
            pltpu.SemaphoreType.DMA,
        ],
        compiler_params=pltpu.CompilerParams(use_tc_tiling_on_sc=False,
                                             needs_layout_passes=False),
    )
    out5 = run(x4, table)
    return out5.transpose(2, 4, 0, 1, 3).reshape(BATCH, SEQ, EMBED_DIM)


# speedup vs baseline: 1.0880x; 1.0880x over previous
import functools

import jax
import jax.numpy as jnp
from jax import lax
from jax.experimental import pallas as pl
from jax.experimental.pallas import tpu as pltpu
from jax.experimental.pallas import tpu_sc as plsc

NUM_HEROES = 1000000
EMBED_DIM = 32
BATCH = 16384
SEQ = 10

_info = plsc.get_sparse_core_info()
NC, NS, NL = _info.num_cores, _info.num_subcores, _info.num_lanes
NW = NC * NS
NBT = BATCH // 128
BT_PER_W = NBT // NW
SEQ_PAD = 16


def _body(x4_hbm, table_hbm, out3, idx_c, buf0, buf1, g0, g1, w0, w1):
    wid = lax.axis_index("s") * NC + lax.axis_index("c")
    bufs = (buf0, buf1)
    gsems = (g0, g1)
    wsems = (w0, w1)

    def stage_idx(c):
        bt = BT_PER_W * wid + c
        pltpu.sync_copy(x4_hbm.at[0, bt], idx_c.at[c, pl.ds(0, 8)])
        pltpu.sync_copy(x4_hbm.at[1, bt, pl.ds(0, 2)], idx_c.at[c, pl.ds(8, 2)])

    def fire_gathers(c, b):
        return [
            pltpu.async_copy(table_hbm.at[idx_c.at[c, s]], bufs[b].at[s],
                             gsems[b])
            for s in range(SEQ)
        ]

    for c in range(BT_PER_W):
        stage_idx(c)

    gd = {}
    wd = {}
    for c in range(BT_PER_W + 1):
        if c < BT_PER_W:
            b = c % 2
            if c >= 2:
                wd[c - 2].wait()
            gd[c] = fire_gathers(c, b)
        if c >= 1:
            k = c - 1
            b = k % 2
            for d in gd[k]:
                d.wait()
            wd[k] = pltpu.async_copy(
                bufs[b],
                out3.at[:, pl.ds((BT_PER_W * wid + k) * 128, 128)],
                wsems[b])
    wd[BT_PER_W - 1].wait()
    wd[BT_PER_W - 2].wait()


@jax.jit
def kernel(x, table):
    xp = jnp.pad(x.T, ((0, SEQ_PAD - SEQ), (0, 0)))
    x4 = xp.reshape(2, 8, NBT, 128).transpose(0, 2, 1, 3)

    run = pl.kernel(
        _body,
        out_type=jax.ShapeDtypeStruct((SEQ, BATCH, EMBED_DIM), jnp.float32),
        mesh=plsc.VectorSubcoreMesh(core_axis_name="c", subcore_axis_name="s"),
        scratch_types=[
            pltpu.VMEM((BT_PER_W, SEQ, 128), jnp.int32),
            pltpu.VMEM((SEQ, 128, EMBED_DIM), jnp.float32),
            pltpu.VMEM((SEQ, 128, EMBED_DIM), jnp.float32),
            pltpu.SemaphoreType.DMA,
            pltpu.SemaphoreType.DMA,
            pltpu.SemaphoreType.DMA,
            pltpu.SemaphoreType.DMA,
        ],
        compiler_params=pltpu.CompilerParams(use_tc_tiling_on_sc=False,
                                             needs_layout_passes=False),
    )
    out3 = run(x4, table)
    return out3.transpose(1, 0, 2)
